# Initial kernel scaffold; baseline (speedup 1.0000x reference)
import jax
import jax.numpy as jnp
from jax.experimental import pallas as pl


def kernel(x, edge_index, edge_weight, W1, b1, g1, be1, W2, b2, g2, be2, wih1, whh1, wch1, bl1, wih2, whh2, wch2, bl2, Wout, bout):
    raise NotImplementedError("write your pallas kernel here")


# speedup vs baseline: 10.0741x; 10.0741x over previous
import functools

import jax
import jax.numpy as jnp
from jax import lax
from jax.experimental import pallas as pl
from jax.experimental.pallas import tpu as pltpu
from jax.experimental.pallas import tpu_sc as plsc

N = 10000
D = 128
H = 128
E = 320000

NC = 2
NS = 16
L = 16
NW = NC * NS
CH = 128
CPT = 79
EPT = CPT * CH
EP = NW * EPT
NP = 10112
SR = NP // NS

_mesh = plsc.VectorSubcoreMesh(core_axis_name="c", subcore_axis_name="s")


@functools.partial(
    pl.kernel,
    out_type=jax.ShapeDtypeStruct((NW, N), jnp.float32),
    mesh=_mesh,
    compiler_params=pltpu.CompilerParams(needs_layout_passes=False),
    scratch_types=[
        pltpu.VMEM((CPT, CH), jnp.int32),
        pltpu.VMEM((CPT, CH), jnp.float32),
        pltpu.VMEM((N,), jnp.float32),
    ],
)
def _deg_kernel(dst_hbm, ew_hbm, out_hbm, dst_v, ew_v, acc_v):
    c = lax.axis_index("c")
    s = lax.axis_index("s")
    wid = c * NS + s
    zeros = jnp.zeros((L,), jnp.float32)

    def zbody(i, _):
        acc_v[pl.ds(i * L, L)] = zeros
        return 0

    lax.fori_loop(0, N // L, zbody, 0)
    pltpu.sync_copy(dst_hbm.at[wid], dst_v)
    pltpu.sync_copy(ew_hbm.at[wid], ew_v)

    def ebody(i, _):
        c0 = i // (CH // L)
        g = i % (CH // L)
        idx = dst_v[c0, pl.ds(g * L, L)]
        w = ew_v[c0, pl.ds(g * L, L)]
        plsc.addupdate_scatter(acc_v, [idx], w)
        return 0

    lax.fori_loop(0, CPT * (CH // L), ebody, 0)
    pltpu.sync_copy(acc_v, out_hbm.at[wid])


@functools.partial(
    pl.kernel,
    out_type=jax.ShapeDtypeStruct((NC, NP, D), jnp.float32),
    mesh=_mesh,
    compiler_params=pltpu.CompilerParams(needs_layout_passes=False),
    scratch_types=[
        pltpu.VMEM((CPT, CH), jnp.int32),
        pltpu.VMEM((CPT, CH), jnp.int32),
        pltpu.VMEM((CPT, CH), jnp.float32),
        pltpu.VMEM((CH, D), jnp.float32),
        pltpu.VMEM_SHARED((NP, D), jnp.float32),
        pltpu.SemaphoreType.DMA,
    ],
)
def _conv_kernel(y_hbm, src_hbm, dst_hbm, ew_hbm, zrows_hbm, out_hbm,
                 src_v, dst_v, ew_v, rows_v, acc_sh, sem):
    c = lax.axis_index("c")
    s = lax.axis_index("s")
    wid = c * NS + s
    stripe = s * SR
    pltpu.sync_copy(zrows_hbm, acc_sh.at[pl.ds(stripe, SR)])
    pltpu.sync_copy(src_hbm.at[wid], src_v)
    pltpu.sync_copy(dst_hbm.at[wid], dst_v)
    pltpu.sync_copy(ew_hbm.at[wid], ew_v)
    plsc.subcore_barrier()

    def chunk_body(c0, _):
        pltpu.async_copy(y_hbm.at[src_v.at[c0]], rows_v, sem).wait()

        def ebody(i, _):
            w = plsc.load_gather(
                ew_v,
                [jnp.full((L,), c0, jnp.int32), jnp.full((L,), i, jnp.int32)],
            )
            for dd in range(D // L):
                sl = pl.ds(dd * L, L)
                rows_v[i, sl] = rows_v[i, sl] * w
            return 0

        lax.fori_loop(0, CH, ebody, 0)
        pltpu.sync_copy(rows_v, acc_sh.at[dst_v.at[c0]], add=True)
        return 0

    lax.fori_loop(0, CPT, chunk_body, 0)
    plsc.subcore_barrier()
    sl = pl.ds(stripe, SR)
    pltpu.sync_copy(acc_sh.at[sl], out_hbm.at[c].at[sl])


def _tc1_body(degp_ref, x_ref, w1_ref, y_ref, dinv_ref):
    deg = jnp.sum(degp_ref[...], axis=0) + 1.0
    dinv = lax.rsqrt(deg)
    xw = jnp.dot(x_ref[...], w1_ref[...], preferred_element_type=jnp.float32)
    y_ref[...] = dinv[:, None] * xw
    dinv_ref[...] = dinv[:, None]


def _tc2_body(acc_ref, y_ref, dinv_ref, b_ref, g_ref, be_ref, w2_ref, y2_ref):
    dinv = dinv_ref[...]
    acc = acc_ref[0, :N, :] + acc_ref[1, :N, :]
    pre = dinv * (acc + y_ref[...]) + b_ref[...]
    m = jnp.mean(pre, axis=0, keepdims=True)
    v = jnp.mean((pre - m) ** 2, axis=0, keepdims=True)
    h = jax.nn.relu((pre - m) * lax.rsqrt(v + 1e-5) * g_ref[...] + be_ref[...])
    y2_ref[...] = dinv * jnp.dot(h, w2_ref[...],
                                 preferred_element_type=jnp.float32)


def _tc3_body(acc_ref, y_ref, dinv_ref, b_ref, g_ref, be_ref, h_ref):
    acc = acc_ref[0, :N, :] + acc_ref[1, :N, :]
    pre = dinv_ref[...] * (acc + y_ref[...]) + b_ref[...]
    m = jnp.mean(pre, axis=0, keepdims=True)
    v = jnp.mean((pre - m) ** 2, axis=0, keepdims=True)
    h_ref[...] = jax.nn.relu(
        (pre - m) * lax.rsqrt(v + 1e-5) * g_ref[...] + be_ref[...])


def _tc4_body(h_ref, wih1_ref, wch1_ref, bl1_ref, wih2_ref, whh2_ref,
              wch2_ref, bl2_ref, wout_ref, bout_ref, out_ref):
    h = h_ref[...]
    f32 = jnp.float32
    g1 = jnp.dot(h, wih1_ref[...], preferred_element_type=f32) + bl1_ref[...]
    i1 = jax.nn.sigmoid(g1[:, :H])
    c1 = jnp.tanh(g1[:, 2 * H:3 * H])
    cy1 = i1 * c1
    wch1 = wch1_ref[...]
    o1 = g1[:, 3 * H:] + jnp.dot(cy1, wch1[:, 2 * H:],
                                 preferred_element_type=f32)
    hy1 = jax.nn.sigmoid(o1) * jnp.tanh(cy1)
    g2 = (jnp.dot(h, wih2_ref[...], preferred_element_type=f32)
          + jnp.dot(hy1, whh2_ref[...], preferred_element_type=f32)
          + bl2_ref[...])
    wch2 = wch2_ref[...]
    cg = g2[:, 2 * H:3 * H] + jnp.dot(cy1, wch2[:, :H],
                                      preferred_element_type=f32)
    i2 = jax.nn.sigmoid(g2[:, :H])
    f2 = jax.nn.sigmoid(g2[:, H:2 * H] + jnp.dot(cy1, wch2[:, H:2 * H],
                                                 preferred_element_type=f32))
    cy2 = f2 * cy1 + i2 * jnp.tanh(cg)
    o2 = g2[:, 3 * H:] + jnp.dot(cy2, wch2[:, 2 * H:],
                                 preferred_element_type=f32)
    hy2 = jax.nn.sigmoid(o2) * jnp.tanh(cy2)
    out_ref[...] = jnp.dot(hy2, wout_ref[...],
                           preferred_element_type=f32) + bout_ref[...]


_tc1 = pl.pallas_call(
    _tc1_body,
    out_shape=(jax.ShapeDtypeStruct((N, D), jnp.float32),
               jax.ShapeDtypeStruct((N, 1), jnp.float32)),
)

_tc2 = pl.pallas_call(
    _tc2_body,
    out_shape=jax.ShapeDtypeStruct((N, D), jnp.float32),
)

_tc3 = pl.pallas_call(
    _tc3_body,
    out_shape=jax.ShapeDtypeStruct((N, D), jnp.float32),
)

_RB = 1000

_tc4 = pl.pallas_call(
    _tc4_body,
    grid=(N // _RB,),
    in_specs=[
        pl.BlockSpec((_RB, H), lambda i: (i, 0)),
        pl.BlockSpec((H, 4 * H), lambda i: (0, 0)),
        pl.BlockSpec((H, 3 * H), lambda i: (0, 0)),
        pl.BlockSpec((1, 4 * H), lambda i: (0, 0)),
        pl.BlockSpec((H, 4 * H), lambda i: (0, 0)),
        pl.BlockSpec((H, 4 * H), lambda i: (0, 0)),
        pl.BlockSpec((H, 3 * H), lambda i: (0, 0)),
        pl.BlockSpec((1, 4 * H), lambda i: (0, 0)),
        pl.BlockSpec((H, 1), lambda i: (0, 0)),
        pl.BlockSpec((1, 1), lambda i: (0, 0)),
    ],
    out_specs=pl.BlockSpec((_RB, 1), lambda i: (i, 0)),
    out_shape=jax.ShapeDtypeStruct((N, 1), jnp.float32),
)


def kernel(x, edge_index, edge_weight, W1, b1, g1, be1, W2, b2, g2, be2,
           wih1, whh1, wch1, bl1, wih2, whh2, wch2, bl2, Wout, bout):
    src = edge_index[0]
    dst = edge_index[1]
    pad = EP - E
    src_r = jnp.concatenate(
        [src, jnp.zeros((pad,), jnp.int32)]).reshape(NW, CPT, CH)
    dst_r = jnp.concatenate(
        [dst, jnp.zeros((pad,), jnp.int32)]).reshape(NW, CPT, CH)
    ew_r = jnp.concatenate(
        [edge_weight, jnp.zeros((pad,), jnp.float32)]).reshape(NW, CPT, CH)

    zrows = jnp.zeros((SR, D), jnp.float32)
    deg_parts = _deg_kernel(dst_r, ew_r)
    y1, dinv = _tc1(deg_parts, x, W1)
    acc1 = _conv_kernel(y1, src_r, dst_r, ew_r, zrows)
    y2 = _tc2(acc1, y1, dinv, b1.reshape(1, D), g1.reshape(1, D),
              be1.reshape(1, D), W2)
    acc2 = _conv_kernel(y2, src_r, dst_r, ew_r, zrows)
    h = _tc3(acc2, y2, dinv, b2.reshape(1, D), g2.reshape(1, D),
             be2.reshape(1, D))
    out = _tc4(h, wih1, wch1, bl1.reshape(1, 4 * H), wih2, whh2, wch2,
               bl2.reshape(1, 4 * H), Wout, bout.reshape(1, 1))
    return jnp.squeeze(out, axis=1)
